# Initial kernel scaffold; baseline (speedup 1.0000x reference)
import jax
import jax.numpy as jnp
from jax.experimental import pallas as pl


def kernel(x, edge_index, batch, W_fc, Wq0, Wk0, Wv0, Wm10, Wm20, bm10, bm20, eps0, Wq1, Wk1, Wv1, Wm11, Wm21, bm11, bm21, eps1, Wq2, Wk2, Wv2, Wm12, Wm22, bm12, bm22, eps2, Wp0, bp0, g0, be0, Wp1, bp1, g1, be1, Wp2, bp2, g2, be2, Wp3, bp3, g3, be3, Wc, bc):
    raise NotImplementedError("write your pallas kernel here")


# speedup vs baseline: 6.1822x; 6.1822x over previous
import functools

import jax
import jax.numpy as jnp
from jax import lax
from jax.experimental import pallas as pl
from jax.experimental.pallas import tpu as pltpu
from jax.experimental.pallas import tpu_sc as plsc

N = 10000
E = 320000
D = 128
H = 128
G = 128
T = 10
ALPHA = 0.2

BR = 1000
GRID = N // BR
NC = 2
NS = 16
NW = NC * NS
NPAD = 10240
CH = 64
NCH = E // CH
TMAX = (NCH + NW - 1) // NW

_HI = lax.Precision.HIGHEST


def _dotT(a, b):
    return lax.dot_general(a, b, (((1,), (1,)), ((), ())),
                           preferred_element_type=jnp.float32, precision=_HI)


def _dot0(a, b):
    return lax.dot_general(a, b, (((0,), (0,)), ((), ())),
                           preferred_element_type=jnp.float32, precision=_HI)


def _init_body(x_ref, b_ref, wfc_ref, h_ref, pool_ref, cnt_ref):
    i = pl.program_id(0)
    x = x_ref[...]
    h_ref[...] = _dotT(x, wfc_ref[...])
    m = (b_ref[...] == lax.broadcasted_iota(jnp.int32, (1, G), 1)).astype(jnp.float32)
    pool_p = _dot0(m, x)
    cnt_p = jnp.sum(m, axis=0, keepdims=True)

    @pl.when(i == 0)
    def _():
        pool_ref[...] = pool_p
        cnt_ref[...] = cnt_p

    @pl.when(i != 0)
    def _():
        pool_ref[...] += pool_p
        cnt_ref[...] += cnt_p


_init_call = pl.pallas_call(
    _init_body,
    grid=(GRID,),
    in_specs=[
        pl.BlockSpec((BR, D), lambda i: (i, 0)),
        pl.BlockSpec((BR, 1), lambda i: (i, 0)),
        pl.BlockSpec((H, D), lambda i: (0, 0)),
    ],
    out_specs=[
        pl.BlockSpec((BR, H), lambda i: (i, 0)),
        pl.BlockSpec((G, D), lambda i: (0, 0)),
        pl.BlockSpec((1, G), lambda i: (0, 0)),
    ],
    out_shape=[
        jax.ShapeDtypeStruct((N, H), jnp.float32),
        jax.ShapeDtypeStruct((G, D), jnp.float32),
        jax.ShapeDtypeStruct((1, G), jnp.float32),
    ],
)


def _qkv_body(h_ref, wq_ref, wk_ref, wv_ref, q_ref, kv_ref):
    h = h_ref[...]
    q_ref[...] = _dotT(h, wq_ref[...])
    kv_ref[:, :H] = _dotT(h, wk_ref[...])
    kv_ref[:, H:] = _dotT(h, wv_ref[...])


_qkv_call = pl.pallas_call(
    _qkv_body,
    grid=(GRID,),
    in_specs=[
        pl.BlockSpec((BR, H), lambda i: (i, 0)),
        pl.BlockSpec((H, H), lambda i: (0, 0)),
        pl.BlockSpec((H, H), lambda i: (0, 0)),
        pl.BlockSpec((H, H), lambda i: (0, 0)),
    ],
    out_specs=[
        pl.BlockSpec((BR, H), lambda i: (i, 0)),
        pl.BlockSpec((BR, 2 * H), lambda i: (i, 0)),
    ],
    out_shape=[
        jax.ShapeDtypeStruct((N, H), jnp.float32),
        jax.ShapeDtypeStruct((N, 2 * H), jnp.float32),
    ],
)


def _edge_body(q_hbm, kv_hbm, src_hbm, dst_hbm, aggr_out, den_out,
               idx_s, idx_d, qrows, kvrows, vsc, denl, dots,
               aggr_sh, sem1, sem2):
    c = lax.axis_index("c")
    s = lax.axis_index("s")
    w = s * NC + c
    zero16 = jnp.zeros((16,), jnp.float32)

    def zden(i, carry):
        denl[pl.ds(i * 16, 16)] = zero16
        return carry

    lax.fori_loop(0, 640, zden, 0)

    def zv(i, carry):
        for r in range(8):
            vsc[i, pl.ds(r * 16, 16)] = zero16
        return carry

    lax.fori_loop(0, CH, zv, 0)

    for t in range(10):
        pltpu.sync_copy(vsc, aggr_sh.at[pl.ds(s * 640 + t * CH, CH)])
    plsc.subcore_barrier()

    def chunk_body(t, carry):
        cid = w + t * NW

        @pl.when(cid < NCH)
        def _():
            off = cid * CH
            pltpu.sync_copy(src_hbm.at[pl.ds(off, CH)], idx_s)
            pltpu.sync_copy(dst_hbm.at[pl.ds(off, CH)], idx_d)
            cp1 = pltpu.async_copy(q_hbm.at[idx_d], qrows, sem1)
            cp2 = pltpu.async_copy(kv_hbm.at[idx_s], kvrows, sem2)
            cp1.wait()
            cp2.wait()

            def group_body(gi, carry2):
                base = gi * 16
                dst16 = idx_d[pl.ds(base, 16)]
                for e in range(16):
                    row = base + e
                    acc = qrows[row, pl.ds(0, 16)] * kvrows[row, pl.ds(0, 16)]
                    for r in range(1, 8):
                        acc = acc + (qrows[row, pl.ds(r * 16, 16)]
                                     * kvrows[row, pl.ds(r * 16, 16)])
                    dots[pl.ds(e * 16, 16)] = acc
                rowbase = lax.broadcasted_iota(jnp.int32, (16,), 0) * 16
                ee = plsc.load_gather(dots, [rowbase])
                for col in range(1, 16):
                    ee = ee + plsc.load_gather(dots, [rowbase + col])
                ee = jnp.where(ee > 0.0, ee, ALPHA * ee)
                ee = jnp.exp(ee)
                plsc.addupdate_scatter(denl, [dst16], ee)
                for e in range(16):
                    row = base + e
                    a = ee[e]
                    for r in range(8):
                        vsc[row, pl.ds(r * 16, 16)] = (
                            kvrows[row, pl.ds(H + r * 16, 16)] * a)
                return carry2

            lax.fori_loop(0, CH // 16, group_body, 0)
            pltpu.sync_copy(vsc, aggr_sh.at[idx_d], add=True)

        return carry

    lax.fori_loop(0, TMAX, chunk_body, 0)

    pltpu.sync_copy(denl, den_out.at[c, s])
    plsc.subcore_barrier()
    for t in range(5):
        pltpu.sync_copy(aggr_sh.at[pl.ds(s * 640 + t * 128, 128)],
                        aggr_out.at[c, pl.ds(s * 640 + t * 128, 128)])


@functools.cache
def _edge_call():
  return pl.kernel(
    _edge_body,
    out_type=[
        jax.ShapeDtypeStruct((NC, NPAD, H), jnp.float32),
        jax.ShapeDtypeStruct((NC, NS, NPAD), jnp.float32),
    ],
    mesh=plsc.VectorSubcoreMesh(core_axis_name="c", subcore_axis_name="s",
                                num_cores=NC, num_subcores=NS),
    compiler_params=pltpu.CompilerParams(needs_layout_passes=False),
    scratch_types=[
        pltpu.VMEM((CH,), jnp.int32),
        pltpu.VMEM((CH,), jnp.int32),
        pltpu.VMEM((CH, H), jnp.float32),
        pltpu.VMEM((CH, 2 * H), jnp.float32),
        pltpu.VMEM((CH, H), jnp.float32),
        pltpu.VMEM((NPAD,), jnp.float32),
        pltpu.VMEM((256,), jnp.float32),
        pltpu.VMEM_SHARED((NPAD, H), jnp.float32),
        pltpu.SemaphoreType.DMA,
        pltpu.SemaphoreType.DMA,
    ],
  )


def _post_body(h_ref, a0_ref, a1_ref, den_ref, eps_ref, wm1_ref, bm1_ref,
               wm2_ref, bm2_ref, b_ref, h2_ref, pool_ref):
    i = pl.program_id(0)
    den = jnp.sum(den_ref[...], axis=1, keepdims=True)
    agg = (a0_ref[...] + a1_ref[...]) / (den + 1e-16)
    hmid = (1.0 + eps_ref[0, 0]) * h_ref[...] + agg
    h1 = jnp.maximum(_dotT(hmid, wm1_ref[...]) + bm1_ref[...], 0.0)
    h2 = jnp.maximum(_dotT(h1, wm2_ref[...]) + bm2_ref[...], 0.0)
    h2_ref[...] = h2
    m = (b_ref[...] == lax.broadcasted_iota(jnp.int32, (1, G), 1)).astype(jnp.float32)
    pool_p = _dot0(m, h2)

    @pl.when(i == 0)
    def _():
        pool_ref[...] = pool_p

    @pl.when(i != 0)
    def _():
        pool_ref[...] += pool_p


_post_call = pl.pallas_call(
    _post_body,
    grid=(GRID,),
    in_specs=[
        pl.BlockSpec((BR, H), lambda i: (i, 0)),
        pl.BlockSpec((BR, H), lambda i: (i, 0)),
        pl.BlockSpec((BR, H), lambda i: (i, 0)),
        pl.BlockSpec((BR, NW), lambda i: (i, 0)),
        pl.BlockSpec((1, 1), lambda i: (0, 0)),
        pl.BlockSpec((H, H), lambda i: (0, 0)),
        pl.BlockSpec((1, H), lambda i: (0, 0)),
        pl.BlockSpec((H, H), lambda i: (0, 0)),
        pl.BlockSpec((1, H), lambda i: (0, 0)),
        pl.BlockSpec((BR, 1), lambda i: (i, 0)),
    ],
    out_specs=[
        pl.BlockSpec((BR, H), lambda i: (i, 0)),
        pl.BlockSpec((G, H), lambda i: (0, 0)),
    ],
    out_shape=[
        jax.ShapeDtypeStruct((N, H), jnp.float32),
        jax.ShapeDtypeStruct((G, H), jnp.float32),
    ],
)


def _readout_body(cnt_ref, s0_ref, s1_ref, s2_ref, s3_ref,
                  g0_ref, b0_ref, g1_ref, b1_ref, g2_ref, b2_ref, g3_ref, b3_ref,
                  wp0_ref, p0_ref, wp1_ref, p1_ref, wp2_ref, p2_ref,
                  wp3_ref, p3_ref, wc_ref, bc_ref, out_ref):
    cinv = 1.0 / jnp.maximum(cnt_ref[...], 1.0)
    score = jnp.zeros((G, H), jnp.float32)
    for s_ref, g_ref, be_ref, wp_ref, bp_ref in (
            (s0_ref, g0_ref, b0_ref, wp0_ref, p0_ref),
            (s1_ref, g1_ref, b1_ref, wp1_ref, p1_ref),
            (s2_ref, g2_ref, b2_ref, wp2_ref, p2_ref),
            (s3_ref, g3_ref, b3_ref, wp3_ref, p3_ref)):
        hh = s_ref[...] * cinv
        m = jnp.mean(hh, axis=0, keepdims=True)
        v = jnp.mean((hh - m) * (hh - m), axis=0, keepdims=True)
        hh = (hh - m) * lax.rsqrt(v + 1e-5) * g_ref[...] + be_ref[...]
        score = score + jnp.maximum(_dotT(hh, wp_ref[...]) + bp_ref[...], 0.0)
    out_ref[...] = _dotT(score, wc_ref[...]) + bc_ref[...]


_spec128 = pl.BlockSpec((G, H), lambda: (0, 0))
_spec1 = pl.BlockSpec((1, H), lambda: (0, 0))
_readout_call = pl.pallas_call(
    _readout_body,
    grid=(),
    in_specs=[pl.BlockSpec((G, 1), lambda: (0, 0))]
    + [_spec128] * 4
    + [_spec1] * 8
    + [_spec128, _spec1] * 4
    + [_spec128, _spec1],
    out_specs=pl.BlockSpec((G, H), lambda: (0, 0)),
    out_shape=jax.ShapeDtypeStruct((G, H), jnp.float32),
)


def kernel(x, edge_index, batch, W_fc,
           Wq0, Wk0, Wv0, Wm10, Wm20, bm10, bm20, eps0,
           Wq1, Wk1, Wv1, Wm11, Wm21, bm11, bm21, eps1,
           Wq2, Wk2, Wv2, Wm12, Wm22, bm12, bm22, eps2,
           Wp0, bp0, g0, be0,
           Wp1, bp1, g1, be1,
           Wp2, bp2, g2, be2,
           Wp3, bp3, g3, be3,
           Wc, bc):
    src = edge_index[0]
    dst = edge_index[1]
    b2d = batch.reshape(N, 1)

    h, pool0, cnt = _init_call(x, b2d, W_fc)

    pools = [pool0]
    layer_params = [
        (Wq0, Wk0, Wv0, Wm10, bm10, Wm20, bm20, eps0),
        (Wq1, Wk1, Wv1, Wm11, bm11, Wm21, bm21, eps1),
        (Wq2, Wk2, Wv2, Wm12, bm12, Wm22, bm22, eps2),
    ]
    for (Wq, Wk, Wv, Wm1, bm1, Wm2, bm2, eps) in layer_params:
        q, kv = _qkv_call(h, Wq, Wk, Wv)
        aggr2, den2 = _edge_call()(q, kv, src, dst)
        den = den2.reshape(NW, NPAD)[:, :N].T
        h, pool = _post_call(h, aggr2[0, :N], aggr2[1, :N], den,
                             eps.reshape(1, 1).astype(jnp.float32),
                             Wm1, bm1.reshape(1, H), Wm2, bm2.reshape(1, H),
                             b2d)
        pools.append(pool)

    Wc_pad = jnp.zeros((H, H), jnp.float32).at[:T].set(Wc)
    bc_pad = jnp.zeros((1, H), jnp.float32).at[0, :T].set(bc)
    out = _readout_call(
        cnt.reshape(G, 1),
        pools[0], pools[1], pools[2], pools[3],
        g0.reshape(1, H), be0.reshape(1, H),
        g1.reshape(1, H), be1.reshape(1, H),
        g2.reshape(1, H), be2.reshape(1, H),
        g3.reshape(1, H), be3.reshape(1, H),
        Wp0, bp0.reshape(1, H),
        Wp1, bp1.reshape(1, H),
        Wp2, bp2.reshape(1, H),
        Wp3, bp3.reshape(1, H),
        Wc_pad, bc_pad)
    return out[:, :T]


# speedup vs baseline: 7.3741x; 1.1928x over previous
import functools

import jax
import jax.numpy as jnp
from jax import lax
from jax.experimental import pallas as pl
from jax.experimental.pallas import tpu as pltpu
from jax.experimental.pallas import tpu_sc as plsc

N = 10000
E = 320000
D = 128
H = 128
G = 128
T = 10
ALPHA = 0.2

BR = 1000
GRID = N // BR
NC = 2
NS = 16
NW = NC * NS
NPAD = 10240
CH = 32
NCH = E // CH
TMAX = (NCH + NW - 1) // NW

_HI = lax.Precision.HIGHEST


def _dotT(a, b):
    return lax.dot_general(a, b, (((1,), (1,)), ((), ())),
                           preferred_element_type=jnp.float32, precision=_HI)


def _dot0(a, b):
    return lax.dot_general(a, b, (((0,), (0,)), ((), ())),
                           preferred_element_type=jnp.float32, precision=_HI)


def _init_body(x_ref, b_ref, wfc_ref, wq_ref, wk_ref, wv_ref,
               h_ref, q_ref, kv_ref, pool_ref, cnt_ref):
    i = pl.program_id(0)
    x = x_ref[...]
    h = _dotT(x, wfc_ref[...])
    h_ref[...] = h
    q_ref[...] = _dotT(h, wq_ref[...])
    kv_ref[:, :H] = _dotT(h, wk_ref[...])
    kv_ref[:, H:] = _dotT(h, wv_ref[...])
    m = (b_ref[...] == lax.broadcasted_iota(jnp.int32, (1, G), 1)).astype(jnp.float32)
    pool_p = _dot0(m, x)
    cnt_p = jnp.sum(m, axis=0, keepdims=True)

    @pl.when(i == 0)
    def _():
        pool_ref[...] = pool_p
        cnt_ref[...] = cnt_p

    @pl.when(i != 0)
    def _():
        pool_ref[...] += pool_p
        cnt_ref[...] += cnt_p


_init_call = pl.pallas_call(
    _init_body,
    grid=(GRID,),
    in_specs=[
        pl.BlockSpec((BR, D), lambda i: (i, 0)),
        pl.BlockSpec((BR, 1), lambda i: (i, 0)),
        pl.BlockSpec((H, D), lambda i: (0, 0)),
        pl.BlockSpec((H, H), lambda i: (0, 0)),
        pl.BlockSpec((H, H), lambda i: (0, 0)),
        pl.BlockSpec((H, H), lambda i: (0, 0)),
    ],
    out_specs=[
        pl.BlockSpec((BR, H), lambda i: (i, 0)),
        pl.BlockSpec((BR, H), lambda i: (i, 0)),
        pl.BlockSpec((BR, 2 * H), lambda i: (i, 0)),
        pl.BlockSpec((G, D), lambda i: (0, 0)),
        pl.BlockSpec((1, G), lambda i: (0, 0)),
    ],
    out_shape=[
        jax.ShapeDtypeStruct((N, H), jnp.float32),
        jax.ShapeDtypeStruct((N, H), jnp.float32),
        jax.ShapeDtypeStruct((N, 2 * H), jnp.float32),
        jax.ShapeDtypeStruct((G, D), jnp.float32),
        jax.ShapeDtypeStruct((1, G), jnp.float32),
    ],
)


def _compute_chunk(qr, kvr, vs, sci, idd, denl, dots):
    def group_body(gi, carry2):
        base = gi * 16
        dst16 = idd[pl.ds(base, 16)]
        for e in range(16):
            row = base + e
            acc = qr[row, pl.ds(0, 16)] * kvr[row, pl.ds(0, 16)]
            for r in range(1, 8):
                acc = acc + (qr[row, pl.ds(r * 16, 16)]
                             * kvr[row, pl.ds(r * 16, 16)])
            dots[pl.ds(e * 16, 16)] = acc
        rowbase = lax.broadcasted_iota(jnp.int32, (16,), 0) * 16
        ee = plsc.load_gather(dots, [rowbase])
        for col in range(1, 16):
            ee = ee + plsc.load_gather(dots, [rowbase + col])
        ee = jnp.where(ee > 0.0, ee, ALPHA * ee)
        ee = jnp.exp(ee)
        plsc.addupdate_scatter(denl, [dst16], ee)
        sci[pl.ds(base, 16)] = dst16
        for e in range(16):
            row = base + e
            a = ee[e]
            for r in range(8):
                vs[row, pl.ds(r * 16, 16)] = kvr[row, pl.ds(H + r * 16, 16)] * a
        return carry2

    lax.fori_loop(0, CH // 16, group_body, 0)


def _edge_body(q_hbm, kv_hbm, src_hbm, dst_hbm, aggr_out, den_out,
               idxs0, idxs1, idxd0, idxd1, sci0, sci1,
               qr0, qr1, kvr0, kvr1, vs0, vs1, denl, dots,
               aggr_sh, isem0, isem1, rsem0, rsem1, ssem0, ssem1):
    c = lax.axis_index("c")
    s = lax.axis_index("s")
    w = s * NC + c
    K = (NCH - w + NW - 1) // NW
    zero16 = jnp.zeros((16,), jnp.float32)

    idxs = [idxs0, idxs1]
    idxd = [idxd0, idxd1]
    sci = [sci0, sci1]
    qr = [qr0, qr1]
    kvr = [kvr0, kvr1]
    vs = [vs0, vs1]
    isem = [isem0, isem1]
    rsem = [rsem0, rsem1]
    ssem = [ssem0, ssem1]

    def zden(i, carry):
        denl[pl.ds(i * 16, 16)] = zero16
        return carry

    lax.fori_loop(0, NPAD // 16, zden, 0)

    def zv(i, carry):
        for r in range(8):
            vs0[i, pl.ds(r * 16, 16)] = zero16
        return carry

    lax.fori_loop(0, CH, zv, 0)

    for t in range(640 // CH):
        pltpu.sync_copy(vs0, aggr_sh.at[pl.ds(s * 640 + t * CH, CH)])
    plsc.subcore_barrier()

    def issue_idx(t, p):
        off = (w + t * NW) * CH
        pltpu.async_copy(src_hbm.at[pl.ds(off, CH)], idxs[p], isem[p])
        pltpu.async_copy(dst_hbm.at[pl.ds(off, CH)], idxd[p], isem[p])

    def wait_idx(p):
        pltpu.make_async_copy(src_hbm.at[pl.ds(0, CH)], idxs[p], isem[p]).wait()
        pltpu.make_async_copy(dst_hbm.at[pl.ds(0, CH)], idxd[p], isem[p]).wait()

    def issue_rows(p):
        pltpu.async_copy(q_hbm.at[idxd[p]], qr[p], rsem[p])
        pltpu.async_copy(kv_hbm.at[idxs[p]], kvr[p], rsem[p])

    def wait_rows(p):
        pltpu.make_async_copy(q_hbm.at[idxd[p]], qr[p], rsem[p]).wait()
        pltpu.make_async_copy(kv_hbm.at[idxs[p]], kvr[p], rsem[p]).wait()

    def issue_scatter(p):
        pltpu.async_copy(vs[p], aggr_sh.at[sci[p]], ssem[p], add=True)

    def wait_scatter(p):
        pltpu.make_async_copy(vs[p], aggr_sh.at[sci[p]], ssem[p]).wait()

    issue_idx(0, 0)
    wait_idx(0)
    issue_rows(0)
    issue_idx(1, 1)

    def loop_body(t2, carry):
        for p in range(2):
            t = t2 * 2 + p

            @pl.when(t + 1 < K)
            def _():
                wait_idx(1 - p)
                issue_rows(1 - p)

            @pl.when(t < K)
            def _():
                wait_rows(p)

            @pl.when(jnp.logical_and(t >= 2, t < K))
            def _():
                wait_scatter(p)

            @pl.when(t < K)
            def _():
                _compute_chunk(qr[p], kvr[p], vs[p], sci[p], idxd[p],
                               denl, dots)
                issue_scatter(p)

            @pl.when(t + 2 < K)
            def _():
                issue_idx(t + 2, p)
        return carry

    lax.fori_loop(0, (TMAX + 2) // 2, loop_body, 0)
    wait_scatter(0)
    wait_scatter(1)

    pltpu.sync_copy(denl, den_out.at[c, s])
    plsc.subcore_barrier()
    for t in range(5):
        pltpu.sync_copy(aggr_sh.at[pl.ds(s * 640 + t * 128, 128)],
                        aggr_out.at[c, pl.ds(s * 640 + t * 128, 128)])


@functools.cache
def _edge_call():
  return pl.kernel(
    _edge_body,
    out_type=[
        jax.ShapeDtypeStruct((NC, NPAD, H), jnp.float32),
        jax.ShapeDtypeStruct((NC, NS, NPAD), jnp.float32),
    ],
    mesh=plsc.VectorSubcoreMesh(core_axis_name="c", subcore_axis_name="s",
                                num_cores=NC, num_subcores=NS),
    compiler_params=pltpu.CompilerParams(needs_layout_passes=False),
    scratch_types=(
        [pltpu.VMEM((CH,), jnp.int32)] * 6
        + [pltpu.VMEM((CH, H), jnp.float32)] * 2
        + [pltpu.VMEM((CH, 2 * H), jnp.float32)] * 2
        + [pltpu.VMEM((CH, H), jnp.float32)] * 2
        + [pltpu.VMEM((NPAD,), jnp.float32)]
        + [pltpu.VMEM((256,), jnp.float32)]
        + [pltpu.VMEM_SHARED((NPAD, H), jnp.float32)]
        + [pltpu.SemaphoreType.DMA] * 6
    ),
  )


def _post_body(h_ref, a0_ref, a1_ref, den_ref, eps_ref, wm1_ref, bm1_ref,
               wm2_ref, bm2_ref, b_ref, wq_ref, wk_ref, wv_ref,
               h2_ref, pool_ref, q_ref, kv_ref):
    i = pl.program_id(0)
    den = jnp.sum(den_ref[...], axis=1, keepdims=True)
    agg = (a0_ref[...] + a1_ref[...]) / (den + 1e-16)
    hmid = (1.0 + eps_ref[0, 0]) * h_ref[...] + agg
    h1 = jnp.maximum(_dotT(hmid, wm1_ref[...]) + bm1_ref[...], 0.0)
    h2 = jnp.maximum(_dotT(h1, wm2_ref[...]) + bm2_ref[...], 0.0)
    h2_ref[...] = h2
    q_ref[...] = _dotT(h2, wq_ref[...])
    kv_ref[:, :H] = _dotT(h2, wk_ref[...])
    kv_ref[:, H:] = _dotT(h2, wv_ref[...])
    m = (b_ref[...] == lax.broadcasted_iota(jnp.int32, (1, G), 1)).astype(jnp.float32)
    pool_p = _dot0(m, h2)

    @pl.when(i == 0)
    def _():
        pool_ref[...] = pool_p

    @pl.when(i != 0)
    def _():
        pool_ref[...] += pool_p


_post_call = pl.pallas_call(
    _post_body,
    grid=(GRID,),
    in_specs=[
        pl.BlockSpec((BR, H), lambda i: (i, 0)),
        pl.BlockSpec((BR, H), lambda i: (i, 0)),
        pl.BlockSpec((BR, H), lambda i: (i, 0)),
        pl.BlockSpec((BR, NW), lambda i: (i, 0)),
        pl.BlockSpec((1, 1), lambda i: (0, 0)),
        pl.BlockSpec((H, H), lambda i: (0, 0)),
        pl.BlockSpec((1, H), lambda i: (0, 0)),
        pl.BlockSpec((H, H), lambda i: (0, 0)),
        pl.BlockSpec((1, H), lambda i: (0, 0)),
        pl.BlockSpec((BR, 1), lambda i: (i, 0)),
        pl.BlockSpec((H, H), lambda i: (0, 0)),
        pl.BlockSpec((H, H), lambda i: (0, 0)),
        pl.BlockSpec((H, H), lambda i: (0, 0)),
    ],
    out_specs=[
        pl.BlockSpec((BR, H), lambda i: (i, 0)),
        pl.BlockSpec((G, H), lambda i: (0, 0)),
        pl.BlockSpec((BR, H), lambda i: (i, 0)),
        pl.BlockSpec((BR, 2 * H), lambda i: (i, 0)),
    ],
    out_shape=[
        jax.ShapeDtypeStruct((N, H), jnp.float32),
        jax.ShapeDtypeStruct((G, H), jnp.float32),
        jax.ShapeDtypeStruct((N, H), jnp.float32),
        jax.ShapeDtypeStruct((N, 2 * H), jnp.float32),
    ],
)


def _readout_body(cnt_ref, s0_ref, s1_ref, s2_ref, s3_ref,
                  g0_ref, b0_ref, g1_ref, b1_ref, g2_ref, b2_ref, g3_ref, b3_ref,
                  wp0_ref, p0_ref, wp1_ref, p1_ref, wp2_ref, p2_ref,
                  wp3_ref, p3_ref, wc_ref, bc_ref, out_ref):
    cinv = 1.0 / jnp.maximum(cnt_ref[...], 1.0)
    score = jnp.zeros((G, H), jnp.float32)
    for s_ref, g_ref, be_ref, wp_ref, bp_ref in (
            (s0_ref, g0_ref, b0_ref, wp0_ref, p0_ref),
            (s1_ref, g1_ref, b1_ref, wp1_ref, p1_ref),
            (s2_ref, g2_ref, b2_ref, wp2_ref, p2_ref),
            (s3_ref, g3_ref, b3_ref, wp3_ref, p3_ref)):
        hh = s_ref[...] * cinv
        m = jnp.mean(hh, axis=0, keepdims=True)
        v = jnp.mean((hh - m) * (hh - m), axis=0, keepdims=True)
        hh = (hh - m) * lax.rsqrt(v + 1e-5) * g_ref[...] + be_ref[...]
        score = score + jnp.maximum(_dotT(hh, wp_ref[...]) + bp_ref[...], 0.0)
    out_ref[...] = _dotT(score, wc_ref[...]) + bc_ref[...]


_spec128 = pl.BlockSpec((G, H), lambda: (0, 0))
_spec1 = pl.BlockSpec((1, H), lambda: (0, 0))
_readout_call = pl.pallas_call(
    _readout_body,
    grid=(),
    in_specs=[pl.BlockSpec((G, 1), lambda: (0, 0))]
    + [_spec128] * 4
    + [_spec1] * 8
    + [_spec128, _spec1] * 4
    + [_spec128, _spec1],
    out_specs=pl.BlockSpec((G, H), lambda: (0, 0)),
    out_shape=jax.ShapeDtypeStruct((G, H), jnp.float32),
)


def kernel(x, edge_index, batch, W_fc,
           Wq0, Wk0, Wv0, Wm10, Wm20, bm10, bm20, eps0,
           Wq1, Wk1, Wv1, Wm11, Wm21, bm11, bm21, eps1,
           Wq2, Wk2, Wv2, Wm12, Wm22, bm12, bm22, eps2,
           Wp0, bp0, g0, be0,
           Wp1, bp1, g1, be1,
           Wp2, bp2, g2, be2,
           Wp3, bp3, g3, be3,
           Wc, bc):
    src = edge_index[0]
    dst = edge_index[1]
    b2d = batch.reshape(N, 1)

    h, q, kv, pool0, cnt = _init_call(x, b2d, W_fc, Wq0, Wk0, Wv0)

    pools = [pool0]
    layer_params = [
        (Wm10, bm10, Wm20, bm20, eps0, Wq1, Wk1, Wv1),
        (Wm11, bm11, Wm21, bm21, eps1, Wq2, Wk2, Wv2),
        (Wm12, bm12, Wm22, bm22, eps2, Wq2, Wk2, Wv2),
    ]
    for (Wm1, bm1, Wm2, bm2, eps, Wqn, Wkn, Wvn) in layer_params:
        aggr2, den2 = _edge_call()(q, kv, src, dst)
        den = den2.reshape(NW, NPAD)[:, :N].T
        h, pool, q, kv = _post_call(h, aggr2[0, :N], aggr2[1, :N], den,
                                    eps.reshape(1, 1).astype(jnp.float32),
                                    Wm1, bm1.reshape(1, H), Wm2,
                                    bm2.reshape(1, H), b2d, Wqn, Wkn, Wvn)
        pools.append(pool)

    Wc_pad = jnp.zeros((H, H), jnp.float32).at[:T].set(Wc)
    bc_pad = jnp.zeros((1, H), jnp.float32).at[0, :T].set(bc)
    out = _readout_call(
        cnt.reshape(G, 1),
        pools[0], pools[1], pools[2], pools[3],
        g0.reshape(1, H), be0.reshape(1, H),
        g1.reshape(1, H), be1.reshape(1, H),
        g2.reshape(1, H), be2.reshape(1, H),
        g3.reshape(1, H), be3.reshape(1, H),
        Wp0, bp0.reshape(1, H),
        Wp1, bp1.reshape(1, H),
        Wp2, bp2.reshape(1, H),
        Wp3, bp3.reshape(1, H),
        Wc_pad, bc_pad)
    return out[:, :T]
